# baseline (speedup 1.0000x reference)
import jax
import jax.numpy as jnp
from jax.experimental import pallas as pl
from jax.experimental.pallas import tpu as pltpu

_B = 16
_S = 64
_R = _S - 1
_D = 1024
_RUNW = _S * _D
_N = _B * _R
_NBUF = 16
_LAG = 8


def _offs(t):
    b, i = divmod(t, _R)
    src = (b * _S * _S + i * (_S + 1) + 1) * _D
    dst = (b * _R * _S + i * _S) * _D
    return src, dst


def kernel(arr):
    B, S2, D = arr.shape
    src1 = arr.reshape(B * S2 * D)

    def body(in_ref, out_ref, *rest):
        bufs = rest[:_NBUF]
        gsems = rest[_NBUF : 2 * _NBUF]
        ssems = rest[2 * _NBUF :]

        def gcopy(t):
            src, _ = _offs(t)
            return pltpu.make_async_copy(
                in_ref.at[pl.ds(src, _RUNW)], bufs[t % _NBUF], gsems[t % _NBUF]
            )

        def scopy(t):
            _, dst = _offs(t)
            return pltpu.make_async_copy(
                bufs[t % _NBUF], out_ref.at[pl.ds(dst, _RUNW)], ssems[t % _NBUF]
            )

        for t in range(_N + _LAG):
            if t < _N:
                if t >= _NBUF:
                    scopy(t - _NBUF).wait()
                gcopy(t).start()
            if t >= _LAG:
                k = t - _LAG
                gcopy(k).wait()
                scopy(k).start()
        for k in range(max(_N - _NBUF, 0), _N):
            scopy(k).wait()

    out1 = pl.pallas_call(
        body,
        in_specs=[pl.BlockSpec(memory_space=pl.ANY)],
        out_specs=pl.BlockSpec(memory_space=pl.ANY),
        out_shape=jax.ShapeDtypeStruct((_B * _R * _S * _D,), jnp.float32),
        scratch_shapes=(
            [pltpu.VMEM((_RUNW,), jnp.float32) for _ in range(_NBUF)]
            + [pltpu.SemaphoreType.DMA for _ in range(2 * _NBUF)]
        ),
    )(src1)
    return out1.reshape(B, _R * _S, D)


# speedup vs baseline: 3.9323x; 3.9323x over previous
import jax
import jax.numpy as jnp
from jax.experimental import pallas as pl
from jax.experimental.pallas import tpu as pltpu

_B = 16
_S = 64
_R = _S - 1
_D = 1024
_DSPLIT = 2
_DW = _D // _DSPLIT


def kernel(arr):
    B, S2, D = arr.shape

    def body(in_ref, out_ref):
        for i in range(_R):
            out_ref[0, i * _S : (i + 1) * _S, :] = in_ref[
                0, i * (_S + 1) + 1 : i * (_S + 1) + 1 + _S, :
            ]

    out = pl.pallas_call(
        body,
        grid=(B, _DSPLIT),
        in_specs=[
            pl.BlockSpec((1, S2, _DW), lambda b, d: (b, 0, d)),
        ],
        out_specs=pl.BlockSpec((1, _R * _S, _DW), lambda b, d: (b, 0, d)),
        out_shape=jax.ShapeDtypeStruct((B, _R * _S, D), jnp.float32),
    )(arr)
    return out
